# baseline (speedup 1.0000x reference)
import jax
import jax.numpy as jnp
from jax import lax
from jax.experimental import pallas as pl
from jax.experimental.pallas import tpu as pltpu
from jax.experimental.pallas import tpu_sc as plsc

_NC = 2
_NS = 16
_NW = _NC * _NS
_CH = 2
_NBUF = 4
_PF = 2
_L = 16


def _sc_body(x_hbm, p_hbm, o_hbm, pos_v, buf_v, *sems):
    row = pos_v.shape[0]
    rpw = x_hbm.shape[0] // _NW
    nch = rpw // _CH
    in_sems, out_sems = sems[:_NBUF], sems[_NBUF:]
    wid = lax.axis_index("s") * _NC + lax.axis_index("c")
    base = wid * rpw

    pltpu.sync_copy(p_hbm, pos_v)

    def in_copy(c, b):
        return pltpu.make_async_copy(
            x_hbm.at[pl.ds(base + c * _CH, _CH)], buf_v.at[b], in_sems[b])

    def out_copy(c, b):
        return pltpu.make_async_copy(
            buf_v.at[b], o_hbm.at[pl.ds(base + c * _CH, _CH)], out_sems[b])

    for b in range(_PF):
        in_copy(b, b).start()

    n_outer = nch // _NBUF

    def step(o, carry):
        for b in range(_NBUF):
            c = o * _NBUF + b
            in_copy(c, b).wait()

            @plsc.parallel_loop(0, row // _L, unroll=8)
            def _(j):
                off = j * _L
                pv = pos_v[pl.ds(off, _L)]
                buf_v[b, 0, pl.ds(off, _L)] = buf_v[b, 0, pl.ds(off, _L)] + pv
                buf_v[b, 1, pl.ds(off, _L)] = buf_v[b, 1, pl.ds(off, _L)] + pv

            out_copy(c, b).start()

            b2 = (b + _PF) % _NBUF

            @pl.when(c + _PF < nch)
            def _():
                @pl.when(c >= _PF)
                def _():
                    out_copy(c - _PF, b2).wait()

                in_copy(c + _PF, b2).start()
        return carry

    lax.fori_loop(0, n_outer, step, 0)

    for c in range(nch - _PF - _PF, nch):
        out_copy(c, c % _NBUF).wait()


def kernel(x, pos_table):
    B, S, D = x.shape
    row = S * D
    x2 = x.reshape(B, row)
    p1 = pos_table.reshape(row)
    mesh = plsc.VectorSubcoreMesh(core_axis_name="c", subcore_axis_name="s")
    out = pl.kernel(
        _sc_body,
        out_type=jax.ShapeDtypeStruct((B, row), jnp.float32),
        mesh=mesh,
        scratch_types=[
            pltpu.VMEM((row,), jnp.float32),
            pltpu.VMEM((_NBUF, _CH, row), jnp.float32),
        ] + [pltpu.SemaphoreType.DMA] * (2 * _NBUF),
    )(x2, p1)
    return out.reshape(B, S, D)
